# baseline (speedup 1.0000x reference)
import functools

import jax
import jax.numpy as jnp
from jax import lax
from jax.experimental import pallas as pl
from jax.experimental.pallas import tpu as pltpu
from jax.experimental.pallas import tpu_sc as plsc

_B = 4096
_L = 50
_LP = 56
_EMB = 128

_info = plsc.get_sparse_core_info()
_NC = _info.num_cores
_NS = _info.num_subcores
_NW = _NC * _NS
_ROWS_W = _B // _NW


@functools.partial(
    pl.kernel,
    mesh=plsc.VectorSubcoreMesh(core_axis_name="c", subcore_axis_name="s"),
    out_type=jax.ShapeDtypeStruct((_B, _LP, _EMB), jnp.float32),
    scratch_types=[
        pltpu.VMEM((_ROWS_W, _LP), jnp.int32),
        pltpu.VMEM((2, _LP, _EMB), jnp.float32),
        pltpu.SemaphoreType.DMA,
        pltpu.SemaphoreType.DMA,
    ],
)
def _sc_gather(idx_hbm, table_hbm, out_hbm, idx_v, rows_v, sem0, sem1):
    wid = lax.axis_index("s") * _NC + lax.axis_index("c")
    wbase = wid * _ROWS_W
    pltpu.sync_copy(idx_hbm.at[wid], idx_v)

    sems = (sem0, sem1)

    def gather(r, b):
        return pltpu.make_async_copy(
            table_hbm.at[idx_v.at[r]], rows_v.at[b], sems[b])

    def writeback(r, b):
        pltpu.sync_copy(rows_v.at[b], out_hbm.at[wbase + r])

    gather(0, 0).start()

    def body(g, carry):
        for b in range(2):
            r = 2 * g + b
            gather(r + 1, 1 - b).start()
            gather(r, b).wait()
            writeback(r, b)
        return carry

    lax.fori_loop(0, (_ROWS_W - 2) // 2, body, 0)

    gather(_ROWS_W - 1, 1).start()
    gather(_ROWS_W - 2, 0).wait()
    writeback(_ROWS_W - 2, 0)
    gather(_ROWS_W - 1, 1).wait()
    writeback(_ROWS_W - 1, 1)


def kernel(batch_word_indexes, batch_char_indexes, word_table):
    del batch_char_indexes
    idx = jnp.pad(batch_word_indexes, ((0, 0), (0, _LP - _L)))
    idx = idx.reshape(_NW, _ROWS_W, _LP)
    out = _sc_gather(idx, word_table)
    return out[:, :_L, :]


# speedup vs baseline: 1.0015x; 1.0015x over previous
import functools

import jax
import jax.numpy as jnp
from jax import lax
from jax.experimental import pallas as pl
from jax.experimental.pallas import tpu as pltpu
from jax.experimental.pallas import tpu_sc as plsc

_B = 4096
_L = 50
_LP = 56
_EMB = 128
_TOTAL = _B * _LP

_info = plsc.get_sparse_core_info()
_NC = _info.num_cores
_NS = _info.num_subcores
_NW = _NC * _NS
_PER_W = _TOTAL // _NW
_CHUNK = 128
_NCHUNK = _PER_W // _CHUNK


@functools.partial(
    pl.kernel,
    mesh=plsc.VectorSubcoreMesh(core_axis_name="c", subcore_axis_name="s"),
    out_type=jax.ShapeDtypeStruct((_TOTAL, _EMB), jnp.float32),
    scratch_types=[
        pltpu.VMEM((_NCHUNK, _CHUNK), jnp.int32),
        pltpu.VMEM((2, _CHUNK, _EMB), jnp.float32),
        pltpu.SemaphoreType.DMA,
        pltpu.SemaphoreType.DMA,
    ],
)
def _sc_gather(idx_hbm, table_hbm, out_hbm, idx_v, rows_v, sem0, sem1):
    wid = lax.axis_index("s") * _NC + lax.axis_index("c")
    base = wid * _PER_W
    pltpu.sync_copy(idx_hbm.at[wid], idx_v)

    sems = (sem0, sem1)

    def gather(j, b):
        return pltpu.make_async_copy(
            table_hbm.at[idx_v.at[j]], rows_v.at[b], sems[b])

    def writeback(j, b):
        pltpu.sync_copy(rows_v.at[b], out_hbm.at[pl.ds(base + j * _CHUNK, _CHUNK)])

    gather(0, 0).start()

    def body(g, carry):
        for b in range(2):
            j = 2 * g + b
            gather(j + 1, 1 - b).start()
            gather(j, b).wait()
            writeback(j, b)
        return carry

    lax.fori_loop(0, (_NCHUNK - 2) // 2, body, 0)

    gather(_NCHUNK - 1, 1).start()
    gather(_NCHUNK - 2, 0).wait()
    writeback(_NCHUNK - 2, 0)
    gather(_NCHUNK - 1, 1).wait()
    writeback(_NCHUNK - 1, 1)


def kernel(batch_word_indexes, batch_char_indexes, word_table):
    del batch_char_indexes
    idx = jnp.pad(batch_word_indexes, ((0, 0), (0, _LP - _L)))
    idx = idx.reshape(_NW, _NCHUNK, _CHUNK)
    out = _sc_gather(idx, word_table)
    return out.reshape(_B, _LP, _EMB)[:, :_L, :]


# speedup vs baseline: 6.5817x; 6.5722x over previous
import functools

import jax
import jax.numpy as jnp
from jax import lax
from jax.experimental import pallas as pl
from jax.experimental.pallas import tpu as pltpu
from jax.experimental.pallas import tpu_sc as plsc

_B = 4096
_L = 50
_LP = 56
_EMB = 128
_TOTAL = _B * _LP

_info = plsc.get_sparse_core_info()
_NC = _info.num_cores
_NS = _info.num_subcores
_NW = _NC * _NS
_PER_W = _TOTAL // _NW
_CHUNK = 128
_NCHUNK = _PER_W // _CHUNK


@functools.partial(
    pl.kernel,
    mesh=plsc.VectorSubcoreMesh(core_axis_name="c", subcore_axis_name="s"),
    out_type=jax.ShapeDtypeStruct((_TOTAL, _EMB), jnp.float32),
    scratch_types=[
        pltpu.VMEM((_NCHUNK, _CHUNK), jnp.int32),
        pltpu.VMEM((2, _CHUNK, _EMB), jnp.float32),
        pltpu.SemaphoreType.DMA,
        pltpu.SemaphoreType.DMA,
    ],
)
def _sc_gather(idx_hbm, table_hbm, out_hbm, idx_v, rows_v, sem0, sem1):
    wid = lax.axis_index("s") * _NC + lax.axis_index("c")
    base = wid * _PER_W
    pltpu.sync_copy(idx_hbm.at[wid], idx_v)

    sems = (sem0, sem1)

    def gather(j, b):
        return pltpu.make_async_copy(
            table_hbm.at[idx_v.at[j]], rows_v.at[b], sems[b])

    def writeback(j, b):
        pltpu.sync_copy(rows_v.at[b], out_hbm.at[pl.ds(base + j * _CHUNK, _CHUNK)])

    gather(0, 0).start()

    def body(g, carry):
        for b in range(2):
            j = 2 * g + b
            gather(j + 1, 1 - b).start()
            gather(j, b).wait()
            writeback(j, b)
        return carry

    lax.fori_loop(0, (_NCHUNK - 2) // 2, body, 0)

    gather(_NCHUNK - 1, 1).start()
    gather(_NCHUNK - 2, 0).wait()
    writeback(_NCHUNK - 2, 0)
    gather(_NCHUNK - 1, 1).wait()
    writeback(_NCHUNK - 1, 1)


def kernel(batch_word_indexes, batch_char_indexes, word_table):
    del batch_char_indexes
    vocab = word_table.shape[0]
    pad = (jax.lax.broadcasted_iota(jnp.int32, (_B, _LP - _L), 0) * (_LP - _L)
           + jax.lax.broadcasted_iota(jnp.int32, (_B, _LP - _L), 1)) % vocab
    idx = jnp.concatenate([batch_word_indexes, pad], axis=1)
    idx = idx.reshape(_NW, _NCHUNK, _CHUNK)
    out = _sc_gather(idx, word_table)
    return out.reshape(_B, _LP, _EMB)[:, :_L, :]
